# baseline (speedup 1.0000x reference)
import jax
import jax.numpy as jnp
from jax import lax
from jax.experimental import pallas as pl
from jax.experimental.pallas import tpu as pltpu
from jax.experimental.pallas import tpu_sc as plsc

N = 10000
D = 128
H = 64
E = 160000
WD = 80
NC, NS = 2, 16
CHUNK = 128
NB = 2
CPW = 80
IH = 20
E_PAD = NS * CPW * CHUNK
ROWS_SP = 10112
RPT = ROWS_SP // NS
NFULL = RPT // CHUNK
REM = RPT % CHUNK
BR = 1000
NPAD = 8


def _proj_body(xu_ref, xt_ref, wlp_ref, wll_ref, wrp_ref, wrl_ref,
               yu_ref, yt_ref, z_ref):
    dn = (((1,), (1,)), ((), ()))
    xu = xu_ref[...]
    xt = xt_ref[...]
    yu = lax.dot_general(xu, wlp_ref[...], dn, preferred_element_type=jnp.float32)
    yt = lax.dot_general(xt, wll_ref[...], dn, preferred_element_type=jnp.float32)
    z = lax.dot_general(xt, wrp_ref[...] + wrl_ref[...], dn,
                        preferred_element_type=jnp.float32)
    ext = (lax.broadcasted_iota(jnp.int32, (xu.shape[0], WD - H), 1) == 0
           ).astype(jnp.float32)
    yu_ref[...] = jnp.concatenate([yu, ext], axis=1)
    yt_ref[...] = jnp.concatenate([yt, ext], axis=1)
    z_ref[...] = z


def _chunked(base):
    for k in range(NFULL):
        yield base + k * CHUNK, CHUNK
    if REM:
        yield base + NFULL * CHUNK, REM


def _sc_body(yu_hbm, yt_hbm, sp_hbm, dp_hbm, sl_hbm, dl_hbm, zrow_hbm,
             out_hbm, ytab, agg, idx_s, idx_d, rows, sem):
    c = lax.axis_index("c")
    s = lax.axis_index("s")
    base = s * RPT
    descs = []
    for off, sz in _chunked(base):
        descs.append(pltpu.async_copy(zrow_hbm.at[pl.ds(0, sz)],
                                      agg.at[pl.ds(off, sz)], sem))
    for y_h, cc in ((yu_hbm, 0), (yt_hbm, 1)):
        @pl.when(c == cc)
        def _stage(y_h=y_h):
            for off, sz in _chunked(base):
                pltpu.async_copy(y_h.at[pl.ds(off, sz)],
                                 ytab.at[pl.ds(off, sz)], sem).wait()
    for d_ in descs:
        d_.wait()
    del descs
    plsc.subcore_barrier()
    for src_h, dst_h, cc in ((sp_hbm, dp_hbm, 0), (sl_hbm, dl_hbm, 1)):
        @pl.when(c == cc)
        def _edges(src_h=src_h, dst_h=dst_h):
            for phase in range(CPW // IH):
                pltpu.sync_copy(src_h.at[s, pl.ds(phase * IH, IH)], idx_s)
                pltpu.sync_copy(dst_h.at[s, pl.ds(phase * IH, IH)], idx_d)
                for b in range(NB):
                    pltpu.async_copy(ytab.at[idx_s.at[b]], rows.at[b], sem)

                def _round(i, carry):
                    for p in range(NB):
                        j = NB * i + p
                        pltpu.make_async_copy(yu_hbm.at[pl.ds(0, CHUNK)],
                                              rows.at[p], sem).wait()
                        pltpu.sync_copy(rows.at[p], agg.at[idx_d.at[j]],
                                        add=True)

                        @pl.when(i < IH // NB - 1)
                        def _refill(j=j, p=p):
                            pltpu.async_copy(ytab.at[idx_s.at[j + NB]],
                                             rows.at[p], sem)
                    return carry

                lax.fori_loop(0, IH // NB, _round, 0)
    plsc.subcore_barrier()
    wdescs = [
        pltpu.async_copy(agg.at[pl.ds(off, sz)],
                         out_hbm.at[c, pl.ds(off, sz)], sem)
        for off, sz in _chunked(base)
    ]
    for d_ in wdescs:
        d_.wait()


def _post_body(parts_ref, z_ref, bsum_ref, wout_ref, bout_ref, tx_ref, lg_ref):
    p = parts_ref[...]
    sp = p[0]
    sl = p[1]
    mp = sp[:, :H] / jnp.maximum(sp[:, H:H + 1], 1.0)
    ml = sl[:, :H] / jnp.maximum(sl[:, H:H + 1], 1.0)
    h = jnp.maximum((mp + ml + z_ref[...] + bsum_ref[...]) * 0.5, 0.0)
    tx_ref[...] = h
    dn = (((1,), (1,)), ((), ()))
    full = lax.dot_general(h, wout_ref[...], dn,
                           preferred_element_type=jnp.float32) + bout_ref[0]
    lg_ref[...] = full[:, :1]


def _prep_edges(ei):
    ei = ei.astype(jnp.int32)
    pad_s = jnp.zeros((E_PAD - E,), jnp.int32)
    pad_d = jnp.full((E_PAD - E,), N, jnp.int32)
    src = jnp.concatenate([ei[0], pad_s]).reshape(NS, CPW, CHUNK)
    dst = jnp.concatenate([ei[1], pad_d]).reshape(NS, CPW, CHUNK)
    return src, dst


def kernel(x_user, x_transaction, edge_index_pays, edge_index_paid_by,
           edge_index_linked, Wl_pays, bl_pays, Wr_pays, Wl_paid_by,
           bl_paid_by, Wr_paid_by, Wl_linked, bl_linked, Wr_linked,
           W_out, b_out):
    f32 = jnp.float32
    grid = N // BR

    yu, yt, z = pl.pallas_call(
        _proj_body,
        grid=(grid,),
        in_specs=[
            pl.BlockSpec((BR, D), lambda i: (i, 0)),
            pl.BlockSpec((BR, D), lambda i: (i, 0)),
            pl.BlockSpec((H, D), lambda i: (0, 0)),
            pl.BlockSpec((H, D), lambda i: (0, 0)),
            pl.BlockSpec((H, D), lambda i: (0, 0)),
            pl.BlockSpec((H, D), lambda i: (0, 0)),
        ],
        out_specs=[
            pl.BlockSpec((BR, WD), lambda i: (i, 0)),
            pl.BlockSpec((BR, WD), lambda i: (i, 0)),
            pl.BlockSpec((BR, H), lambda i: (i, 0)),
        ],
        out_shape=[
            jax.ShapeDtypeStruct((ROWS_SP, WD), f32),
            jax.ShapeDtypeStruct((ROWS_SP, WD), f32),
            jax.ShapeDtypeStruct((N, H), f32),
        ],
    )(x_user.astype(f32), x_transaction.astype(f32),
      Wl_pays, Wl_linked, Wr_pays, Wr_linked)

    sp_, dp_ = _prep_edges(edge_index_pays)
    sl_, dl_ = _prep_edges(edge_index_linked)
    zrow = jnp.zeros((CHUNK, WD), f32)

    mesh = plsc.VectorSubcoreMesh(core_axis_name="c", subcore_axis_name="s",
                                  num_cores=NC, num_subcores=NS)
    parts = pl.kernel(
        _sc_body,
        jax.ShapeDtypeStruct((NC, ROWS_SP, WD), f32),
        mesh=mesh,
        scratch_types=[
            pltpu.VMEM_SHARED((ROWS_SP, WD), f32),
            pltpu.VMEM_SHARED((ROWS_SP, WD), f32),
            pltpu.VMEM((IH, CHUNK), jnp.int32),
            pltpu.VMEM((IH, CHUNK), jnp.int32),
            pltpu.VMEM((NB, CHUNK, WD), f32),
            pltpu.SemaphoreType.DMA,
        ],
        compiler_params=pltpu.CompilerParams(use_tc_tiling_on_sc=False),
    )(yu, yt, sp_, dp_, sl_, dl_, zrow)

    tx, lg = pl.pallas_call(
        _post_body,
        grid=(grid,),
        in_specs=[
            pl.BlockSpec((NC, BR, WD), lambda i: (0, i, 0)),
            pl.BlockSpec((BR, H), lambda i: (i, 0)),
            pl.BlockSpec((1, H), lambda i: (0, 0)),
            pl.BlockSpec((NPAD, H), lambda i: (0, 0)),
            pl.BlockSpec(memory_space=pltpu.SMEM),
        ],
        out_specs=[
            pl.BlockSpec((BR, H), lambda i: (i, 0)),
            pl.BlockSpec((BR, 1), lambda i: (i, 0)),
        ],
        out_shape=[
            jax.ShapeDtypeStruct((N, H), f32),
            jax.ShapeDtypeStruct((N, 1), f32),
        ],
    )(parts, z, (bl_pays + bl_linked).reshape(1, H),
      jnp.zeros((NPAD, H), f32).at[0].set(W_out[0]), b_out)

    return lg.reshape(N), tx


# speedup vs baseline: 1.0205x; 1.0205x over previous
import jax
import jax.numpy as jnp
from jax import lax
from jax.experimental import pallas as pl
from jax.experimental.pallas import tpu as pltpu
from jax.experimental.pallas import tpu_sc as plsc

N = 10000
D = 128
H = 64
E = 160000
WD = 72
NC, NS = 2, 16
CHUNK = 128
NB = 2
CPW = 80
IH = 20
E_PAD = NS * CPW * CHUNK
ROWS_SP = 10112
RPT = ROWS_SP // NS
NFULL = RPT // CHUNK
REM = RPT % CHUNK
BR = 1000
NPAD = 8


def _proj_body(xu_ref, xt_ref, wlp_ref, wll_ref, wrp_ref, wrl_ref,
               yu_ref, yt_ref, z_ref):
    dn = (((1,), (1,)), ((), ()))
    xu = xu_ref[...]
    xt = xt_ref[...]
    yu = lax.dot_general(xu, wlp_ref[...], dn, preferred_element_type=jnp.float32)
    yt = lax.dot_general(xt, wll_ref[...], dn, preferred_element_type=jnp.float32)
    z = lax.dot_general(xt, wrp_ref[...] + wrl_ref[...], dn,
                        preferred_element_type=jnp.float32)
    ext = (lax.broadcasted_iota(jnp.int32, (xu.shape[0], WD - H), 1) == 0
           ).astype(jnp.float32)
    yu_ref[...] = jnp.concatenate([yu, ext], axis=1)
    yt_ref[...] = jnp.concatenate([yt, ext], axis=1)
    z_ref[...] = z


def _chunked(base):
    for k in range(NFULL):
        yield base + k * CHUNK, CHUNK
    if REM:
        yield base + NFULL * CHUNK, REM


def _sc_body(yu_hbm, yt_hbm, sp_hbm, dp_hbm, sl_hbm, dl_hbm, zrow_hbm,
             out_hbm, ytab, agg, idx_s, idx_d, rows, sem):
    c = lax.axis_index("c")
    s = lax.axis_index("s")
    base = s * RPT
    descs = []
    for off, sz in _chunked(base):
        descs.append(pltpu.async_copy(zrow_hbm.at[pl.ds(0, sz)],
                                      agg.at[pl.ds(off, sz)], sem))
    for y_h, cc in ((yu_hbm, 0), (yt_hbm, 1)):
        @pl.when(c == cc)
        def _stage(y_h=y_h):
            for off, sz in _chunked(base):
                pltpu.async_copy(y_h.at[pl.ds(off, sz)],
                                 ytab.at[pl.ds(off, sz)], sem).wait()
    for d_ in descs:
        d_.wait()
    del descs
    plsc.subcore_barrier()
    for src_h, dst_h, cc in ((sp_hbm, dp_hbm, 0), (sl_hbm, dl_hbm, 1)):
        @pl.when(c == cc)
        def _edges(src_h=src_h, dst_h=dst_h):
            for phase in range(CPW // IH):
                pltpu.sync_copy(src_h.at[s, pl.ds(phase * IH, IH)], idx_s)
                pltpu.sync_copy(dst_h.at[s, pl.ds(phase * IH, IH)], idx_d)
                for b in range(NB):
                    pltpu.async_copy(ytab.at[idx_s.at[b]], rows.at[b], sem)

                def _round(i, carry):
                    for p in range(NB):
                        j = NB * i + p
                        pltpu.make_async_copy(yu_hbm.at[pl.ds(0, CHUNK)],
                                              rows.at[p], sem).wait()
                        pltpu.sync_copy(rows.at[p], agg.at[idx_d.at[j]],
                                        add=True)

                        @pl.when(i < IH // NB - 1)
                        def _refill(j=j, p=p):
                            pltpu.async_copy(ytab.at[idx_s.at[j + NB]],
                                             rows.at[p], sem)
                    return carry

                lax.fori_loop(0, IH // NB, _round, 0)
    plsc.subcore_barrier()
    wdescs = [
        pltpu.async_copy(agg.at[pl.ds(off, sz)],
                         out_hbm.at[c, pl.ds(off, sz)], sem)
        for off, sz in _chunked(base)
    ]
    for d_ in wdescs:
        d_.wait()


def _post_body(parts_ref, z_ref, bsum_ref, wout_ref, bout_ref, tx_ref, lg_ref):
    p = parts_ref[...]
    sp = p[0]
    sl = p[1]
    mp = sp[:, :H] / jnp.maximum(sp[:, H:H + 1], 1.0)
    ml = sl[:, :H] / jnp.maximum(sl[:, H:H + 1], 1.0)
    h = jnp.maximum((mp + ml + z_ref[...] + bsum_ref[...]) * 0.5, 0.0)
    tx_ref[...] = h
    dn = (((1,), (1,)), ((), ()))
    full = lax.dot_general(h, wout_ref[...], dn,
                           preferred_element_type=jnp.float32) + bout_ref[0]
    lg_ref[...] = full[:, :1]


def _prep_edges(ei):
    ei = ei.astype(jnp.int32)
    pad_s = jnp.zeros((E_PAD - E,), jnp.int32)
    pad_d = jnp.full((E_PAD - E,), N, jnp.int32)
    src = jnp.concatenate([ei[0], pad_s]).reshape(NS, CPW, CHUNK)
    dst = jnp.concatenate([ei[1], pad_d]).reshape(NS, CPW, CHUNK)
    return src, dst


def kernel(x_user, x_transaction, edge_index_pays, edge_index_paid_by,
           edge_index_linked, Wl_pays, bl_pays, Wr_pays, Wl_paid_by,
           bl_paid_by, Wr_paid_by, Wl_linked, bl_linked, Wr_linked,
           W_out, b_out):
    f32 = jnp.float32
    grid = N // BR

    yu, yt, z = pl.pallas_call(
        _proj_body,
        grid=(grid,),
        in_specs=[
            pl.BlockSpec((BR, D), lambda i: (i, 0)),
            pl.BlockSpec((BR, D), lambda i: (i, 0)),
            pl.BlockSpec((H, D), lambda i: (0, 0)),
            pl.BlockSpec((H, D), lambda i: (0, 0)),
            pl.BlockSpec((H, D), lambda i: (0, 0)),
            pl.BlockSpec((H, D), lambda i: (0, 0)),
        ],
        out_specs=[
            pl.BlockSpec((BR, WD), lambda i: (i, 0)),
            pl.BlockSpec((BR, WD), lambda i: (i, 0)),
            pl.BlockSpec((BR, H), lambda i: (i, 0)),
        ],
        out_shape=[
            jax.ShapeDtypeStruct((ROWS_SP, WD), f32),
            jax.ShapeDtypeStruct((ROWS_SP, WD), f32),
            jax.ShapeDtypeStruct((N, H), f32),
        ],
    )(x_user.astype(f32), x_transaction.astype(f32),
      Wl_pays, Wl_linked, Wr_pays, Wr_linked)

    sp_, dp_ = _prep_edges(edge_index_pays)
    sl_, dl_ = _prep_edges(edge_index_linked)
    zrow = jnp.zeros((CHUNK, WD), f32)

    mesh = plsc.VectorSubcoreMesh(core_axis_name="c", subcore_axis_name="s",
                                  num_cores=NC, num_subcores=NS)
    parts = pl.kernel(
        _sc_body,
        jax.ShapeDtypeStruct((NC, ROWS_SP, WD), f32),
        mesh=mesh,
        scratch_types=[
            pltpu.VMEM_SHARED((ROWS_SP, WD), f32),
            pltpu.VMEM_SHARED((ROWS_SP, WD), f32),
            pltpu.VMEM((IH, CHUNK), jnp.int32),
            pltpu.VMEM((IH, CHUNK), jnp.int32),
            pltpu.VMEM((NB, CHUNK, WD), f32),
            pltpu.SemaphoreType.DMA,
        ],
        compiler_params=pltpu.CompilerParams(use_tc_tiling_on_sc=False),
    )(yu, yt, sp_, dp_, sl_, dl_, zrow)

    tx, lg = pl.pallas_call(
        _post_body,
        grid=(grid,),
        in_specs=[
            pl.BlockSpec((NC, BR, WD), lambda i: (0, i, 0)),
            pl.BlockSpec((BR, H), lambda i: (i, 0)),
            pl.BlockSpec((1, H), lambda i: (0, 0)),
            pl.BlockSpec((NPAD, H), lambda i: (0, 0)),
            pl.BlockSpec(memory_space=pltpu.SMEM),
        ],
        out_specs=[
            pl.BlockSpec((BR, H), lambda i: (i, 0)),
            pl.BlockSpec((BR, 1), lambda i: (i, 0)),
        ],
        out_shape=[
            jax.ShapeDtypeStruct((N, H), f32),
            jax.ShapeDtypeStruct((N, 1), f32),
        ],
    )(parts, z, (bl_pays + bl_linked).reshape(1, H),
      jnp.zeros((NPAD, H), f32).at[0].set(W_out[0]), b_out)

    return lg.reshape(N), tx


# speedup vs baseline: 1.0960x; 1.0741x over previous
import jax
import jax.numpy as jnp
from jax import lax
from jax.experimental import pallas as pl
from jax.experimental.pallas import tpu as pltpu
from jax.experimental.pallas import tpu_sc as plsc

N = 10000
D = 128
H = 64
E = 160000
WD = 72
NC, NS = 2, 16
CHUNK = 128
NB = 2
NBUF = 4
CPW = 80
IH = 10
E_PAD = NS * CPW * CHUNK
ROWS_SP = 10112
RPT = ROWS_SP // NS
NFULL = RPT // CHUNK
REM = RPT % CHUNK
BR = 1000
NPAD = 8


def _proj_body(xu_ref, xt_ref, wlp_ref, wll_ref, wrp_ref, wrl_ref,
               yu_ref, yt_ref, z_ref):
    dn = (((1,), (1,)), ((), ()))
    xu = xu_ref[...]
    xt = xt_ref[...]
    yu = lax.dot_general(xu, wlp_ref[...], dn, preferred_element_type=jnp.float32)
    yt = lax.dot_general(xt, wll_ref[...], dn, preferred_element_type=jnp.float32)
    z = lax.dot_general(xt, wrp_ref[...] + wrl_ref[...], dn,
                        preferred_element_type=jnp.float32)
    ext = (lax.broadcasted_iota(jnp.int32, (xu.shape[0], WD - H), 1) == 0
           ).astype(jnp.float32)
    yu_ref[...] = jnp.concatenate([yu, ext], axis=1)
    yt_ref[...] = jnp.concatenate([yt, ext], axis=1)
    z_ref[...] = z


def _chunked(base):
    for k in range(NFULL):
        yield base + k * CHUNK, CHUNK
    if REM:
        yield base + NFULL * CHUNK, REM


def _sc_body(yu_hbm, yt_hbm, sp_hbm, dp_hbm, sl_hbm, dl_hbm, zrow_hbm,
             out_hbm, ytab, agg, idx_s, idx_d, rows, sem, sem2):
    c = lax.axis_index("c")
    s = lax.axis_index("s")
    base = s * RPT
    descs = []
    for off, sz in _chunked(base):
        descs.append(pltpu.async_copy(zrow_hbm.at[pl.ds(0, sz)],
                                      agg.at[pl.ds(off, sz)], sem))
    for y_h, cc in ((yu_hbm, 0), (yt_hbm, 1)):
        @pl.when(c == cc)
        def _stage(y_h=y_h):
            for off, sz in _chunked(base):
                pltpu.async_copy(y_h.at[pl.ds(off, sz)],
                                 ytab.at[pl.ds(off, sz)], sem).wait()
    for d_ in descs:
        d_.wait()
    del descs
    plsc.subcore_barrier()
    for src_h, dst_h, cc in ((sp_hbm, dp_hbm, 0), (sl_hbm, dl_hbm, 1)):
        @pl.when(c == cc)
        def _edges(src_h=src_h, dst_h=dst_h):
            def _phase(ph, carry):
                pltpu.sync_copy(src_h.at[s, pl.ds(ph * IH, IH)], idx_s)
                pltpu.sync_copy(dst_h.at[s, pl.ds(ph * IH, IH)], idx_d)
                gd = {b: pltpu.async_copy(ytab.at[idx_s.at[b]], rows.at[b],
                                          sem)
                      for b in range(2)}
                sd = {}
                for j in range(IH):
                    gd[j].wait()
                    sd[j] = pltpu.async_copy(rows.at[j % NBUF],
                                             agg.at[idx_d.at[j]], sem2,
                                             add=True)
                    if j >= 2:
                        sd[j - 2].wait()
                    if j + 2 < IH:
                        gd[j + 2] = pltpu.async_copy(
                            ytab.at[idx_s.at[j + 2]],
                            rows.at[(j + 2) % NBUF], sem)
                sd[IH - 2].wait()
                sd[IH - 1].wait()
                return carry

            lax.fori_loop(0, CPW // IH, _phase, 0)
    plsc.subcore_barrier()
    wdescs = [
        pltpu.async_copy(agg.at[pl.ds(off, sz)],
                         out_hbm.at[c, pl.ds(off, sz)], sem)
        for off, sz in _chunked(base)
    ]
    for d_ in wdescs:
        d_.wait()


def _post_body(parts_ref, z_ref, bsum_ref, wout_ref, bout_ref, tx_ref, lg_ref):
    p = parts_ref[...]
    sp = p[0]
    sl = p[1]
    mp = sp[:, :H] / jnp.maximum(sp[:, H:H + 1], 1.0)
    ml = sl[:, :H] / jnp.maximum(sl[:, H:H + 1], 1.0)
    h = jnp.maximum((mp + ml + z_ref[...] + bsum_ref[...]) * 0.5, 0.0)
    tx_ref[...] = h
    dn = (((1,), (1,)), ((), ()))
    full = lax.dot_general(h, wout_ref[...], dn,
                           preferred_element_type=jnp.float32) + bout_ref[0]
    lg_ref[...] = full[:, :1]


def _prep_edges(ei):
    ei = ei.astype(jnp.int32)
    pad_s = jnp.zeros((E_PAD - E,), jnp.int32)
    pad_d = jnp.full((E_PAD - E,), N, jnp.int32)
    src = jnp.concatenate([ei[0], pad_s]).reshape(NS, CPW, CHUNK)
    dst = jnp.concatenate([ei[1], pad_d]).reshape(NS, CPW, CHUNK)
    return src, dst


def kernel(x_user, x_transaction, edge_index_pays, edge_index_paid_by,
           edge_index_linked, Wl_pays, bl_pays, Wr_pays, Wl_paid_by,
           bl_paid_by, Wr_paid_by, Wl_linked, bl_linked, Wr_linked,
           W_out, b_out):
    f32 = jnp.float32
    grid = N // BR

    yu, yt, z = pl.pallas_call(
        _proj_body,
        grid=(grid,),
        in_specs=[
            pl.BlockSpec((BR, D), lambda i: (i, 0)),
            pl.BlockSpec((BR, D), lambda i: (i, 0)),
            pl.BlockSpec((H, D), lambda i: (0, 0)),
            pl.BlockSpec((H, D), lambda i: (0, 0)),
            pl.BlockSpec((H, D), lambda i: (0, 0)),
            pl.BlockSpec((H, D), lambda i: (0, 0)),
        ],
        out_specs=[
            pl.BlockSpec((BR, WD), lambda i: (i, 0)),
            pl.BlockSpec((BR, WD), lambda i: (i, 0)),
            pl.BlockSpec((BR, H), lambda i: (i, 0)),
        ],
        out_shape=[
            jax.ShapeDtypeStruct((ROWS_SP, WD), f32),
            jax.ShapeDtypeStruct((ROWS_SP, WD), f32),
            jax.ShapeDtypeStruct((N, H), f32),
        ],
    )(x_user.astype(f32), x_transaction.astype(f32),
      Wl_pays, Wl_linked, Wr_pays, Wr_linked)

    sp_, dp_ = _prep_edges(edge_index_pays)
    sl_, dl_ = _prep_edges(edge_index_linked)
    zrow = jnp.zeros((CHUNK, WD), f32)

    mesh = plsc.VectorSubcoreMesh(core_axis_name="c", subcore_axis_name="s",
                                  num_cores=NC, num_subcores=NS)
    parts = pl.kernel(
        _sc_body,
        jax.ShapeDtypeStruct((NC, ROWS_SP, WD), f32),
        mesh=mesh,
        scratch_types=[
            pltpu.VMEM_SHARED((ROWS_SP, WD), f32),
            pltpu.VMEM_SHARED((ROWS_SP, WD), f32),
            pltpu.VMEM((IH, CHUNK), jnp.int32),
            pltpu.VMEM((IH, CHUNK), jnp.int32),
            pltpu.VMEM((NBUF, CHUNK, WD), f32),
            pltpu.SemaphoreType.DMA,
            pltpu.SemaphoreType.DMA,
        ],
        compiler_params=pltpu.CompilerParams(use_tc_tiling_on_sc=False),
    )(yu, yt, sp_, dp_, sl_, dl_, zrow)

    tx, lg = pl.pallas_call(
        _post_body,
        grid=(grid,),
        in_specs=[
            pl.BlockSpec((NC, BR, WD), lambda i: (0, i, 0)),
            pl.BlockSpec((BR, H), lambda i: (i, 0)),
            pl.BlockSpec((1, H), lambda i: (0, 0)),
            pl.BlockSpec((NPAD, H), lambda i: (0, 0)),
            pl.BlockSpec(memory_space=pltpu.SMEM),
        ],
        out_specs=[
            pl.BlockSpec((BR, H), lambda i: (i, 0)),
            pl.BlockSpec((BR, 1), lambda i: (i, 0)),
        ],
        out_shape=[
            jax.ShapeDtypeStruct((N, H), f32),
            jax.ShapeDtypeStruct((N, 1), f32),
        ],
    )(parts, z, (bl_pays + bl_linked).reshape(1, H),
      jnp.zeros((NPAD, H), f32).at[0].set(W_out[0]), b_out)

    return lg.reshape(N), tx
